# baseline (speedup 1.0000x reference)
import functools

import jax
import jax.numpy as jnp
from jax import lax
from jax.experimental import pallas as pl
from jax.experimental.pallas import tpu as pltpu
from jax.experimental.pallas import tpu_sc as plsc

B = 16384
S = 50
EMB = 128
HID = 256

NC, NS = 2, 16
NW = NC * NS
NCHUNK = 2
CB = B // NCHUNK
EPW = CB // NW
G = 1
NGROUPS = EPW // G
GPW = NGROUPS
NVR = EMB // 16
RU = 10
NBUF = 8
DEPTH = 6


def _sc_pool_body(idx_hbm, table_hbm, out_hbm, idx_all, *scratch):
    rows = scratch[0:NBUF]
    outs = scratch[NBUF:2 * NBUF]
    sgs = scratch[2 * NBUF:3 * NBUF]
    sos = scratch[3 * NBUF:4 * NBUF]
    wid = lax.axis_index("s") * NC + lax.axis_index("c")
    ebase = wid * EPW
    pltpu.sync_copy(idx_hbm.at[pl.ds(wid * GPW, GPW)], idx_all)

    def prefetch(g, p):
        pltpu.async_copy(table_hbm.at[idx_all.at[g]], rows[p], sgs[p])

    def consume(i, g, p):
        pltpu.make_async_copy(
            table_hbm.at[idx_all.at[g]], rows[p], sgs[p]).wait()

        @pl.when(i >= 1)
        def _():
            pltpu.make_async_copy(
                outs[p], out_hbm.at[pl.ds(0, G)], sos[p]).wait()

        for e in range(G):
            def row_body(rr, accs, e=e):
                base = e * S + rr * RU
                for k in range(RU):
                    accs = tuple(
                        accs[v] + rows[p][base + k, pl.ds(16 * v, 16)]
                        for v in range(NVR))
                return accs
            accs = lax.fori_loop(
                0, S // RU, row_body,
                tuple(jnp.zeros((16,), jnp.float32) for _ in range(NVR)))
            for v in range(NVR):
                outs[p][e, pl.ds(16 * v, 16)] = accs[v] * (1.0 / S)
        pltpu.async_copy(outs[p], out_hbm.at[pl.ds(ebase + g * G, G)], sos[p])

    for d in range(DEPTH):
        prefetch(d, d)

    def block(i, carry):
        for p in range(NBUF):
            g = NBUF * i + p

            @pl.when(g + DEPTH < NGROUPS)
            def _(p=p, g=g):
                prefetch(g + DEPTH, (p + DEPTH) % NBUF)

            consume(i, g, p)
        return carry

    lax.fori_loop(0, NGROUPS // NBUF, block, 0)
    for p in range(NBUF):
        pltpu.make_async_copy(outs[p], out_hbm.at[pl.ds(0, G)], sos[p]).wait()


_sc_pool = pl.kernel(
    _sc_pool_body,
    out_type=jax.ShapeDtypeStruct((CB, EMB), jnp.float32),
    mesh=plsc.VectorSubcoreMesh(core_axis_name="c", subcore_axis_name="s"),
    scratch_types=(
        [pltpu.VMEM((GPW, G * S), jnp.int32)]
        + [pltpu.VMEM((G * S, EMB), jnp.float32) for _ in range(NBUF)]
        + [pltpu.VMEM((G, EMB), jnp.float32) for _ in range(NBUF)]
        + [pltpu.SemaphoreType.DMA for _ in range(2 * NBUF)]
    ),
)


def _mlp_body(x_ref, w1_ref, b1_ref, w2_ref, b2_ref, o_ref):
    x = x_ref[...]
    h = jnp.dot(x, w1_ref[...], preferred_element_type=jnp.float32)
    h = jnp.maximum(h + b1_ref[...], 0.0)
    o = jnp.dot(h, w2_ref[...], preferred_element_type=jnp.float32)
    o_ref[...] = jax.nn.sigmoid(o + b2_ref[...])[:, :1]


def _mlp(x, w1, b1, w2, b2):
    BM = 2048
    grid = (CB // BM,)
    return pl.pallas_call(
        _mlp_body,
        out_shape=jax.ShapeDtypeStruct((CB, 1), jnp.float32),
        grid=grid,
        in_specs=[
            pl.BlockSpec((BM, EMB), lambda i: (i, 0)),
            pl.BlockSpec((EMB, HID), lambda i: (0, 0)),
            pl.BlockSpec((1, HID), lambda i: (0, 0)),
            pl.BlockSpec((HID, 128), lambda i: (0, 0)),
            pl.BlockSpec((1, 128), lambda i: (0, 0)),
        ],
        out_specs=pl.BlockSpec((BM, 1), lambda i: (i, 0)),
    )(x, w1, b1, w2, b2)


def kernel(inputs, table, W1, b1, W2, b2):
    idx = inputs.astype(jnp.int32).reshape(B // G, G * S)
    w2p = jnp.pad(W2, ((0, 0), (0, 128 - W2.shape[1])))
    b2p = jnp.pad(b2, (0, 128 - b2.shape[0])).reshape(1, 128)
    b1r = b1.reshape(1, HID)
    outs = []
    for c in range(NCHUNK):
        pooled = _sc_pool(idx[c * CB // G:(c + 1) * CB // G], table)
        outs.append(_mlp(pooled, W1, b1r, w2p, b2p))
    return jnp.concatenate(outs, axis=0)


# speedup vs baseline: 1.0307x; 1.0307x over previous
import functools

import jax
import jax.numpy as jnp
from jax import lax
from jax.experimental import pallas as pl
from jax.experimental.pallas import tpu as pltpu
from jax.experimental.pallas import tpu_sc as plsc

B = 16384
S = 50
EMB = 128
HID = 256

NC, NS = 2, 16
NW = NC * NS
NCHUNK = 1
CB = B // NCHUNK
EPW = CB // NW
G = 1
NGROUPS = EPW // G
GPW = NGROUPS
NVR = EMB // 16
RU = 10
NBUF = 8
DEPTH = 7


def _sc_pool_body(idx_hbm, table_hbm, out_hbm, idx_all, rows_all, outs_all,
                  *sems):
    rows = [rows_all.at[pl.ds(p * G * S, G * S)] for p in range(NBUF)]
    outs = [outs_all.at[pl.ds(p * G, G)] for p in range(NBUF)]
    sgs = sems[0:NBUF]
    sos = sems[NBUF:2 * NBUF]
    wid = lax.axis_index("s") * NC + lax.axis_index("c")
    ebase = wid * EPW
    pltpu.sync_copy(idx_hbm.at[pl.ds(wid * GPW, GPW)], idx_all)

    def prefetch(g, p):
        pltpu.async_copy(table_hbm.at[idx_all.at[g]], rows[p], sgs[p])

    def consume(i, g, p):
        pltpu.make_async_copy(
            table_hbm.at[idx_all.at[g]], rows[p], sgs[p]).wait()

        @pl.when(i >= 1)
        def _():
            pltpu.make_async_copy(
                outs[p], out_hbm.at[pl.ds(0, G)], sos[p]).wait()

        for e in range(G):
            def row_body(rr, accs, e=e):
                base = (p * G + e) * S + rr * RU
                for k in range(RU):
                    accs = tuple(
                        accs[v] + rows_all[base + k, pl.ds(16 * v, 16)]
                        for v in range(NVR))
                return accs
            accs = lax.fori_loop(
                0, S // RU, row_body,
                tuple(jnp.zeros((16,), jnp.float32) for _ in range(NVR)))
            for v in range(NVR):
                outs_all[p * G + e, pl.ds(16 * v, 16)] = accs[v] * (1.0 / S)
        pltpu.async_copy(outs[p], out_hbm.at[pl.ds(ebase + g * G, G)], sos[p])

    for d in range(DEPTH):
        prefetch(d, d)

    def block(i, carry):
        for p in range(NBUF):
            g = NBUF * i + p

            @pl.when(g + DEPTH < NGROUPS)
            def _(p=p, g=g):
                prefetch(g + DEPTH, (p + DEPTH) % NBUF)

            consume(i, g, p)
        return carry

    lax.fori_loop(0, NGROUPS // NBUF, block, 0)
    for p in range(NBUF):
        pltpu.make_async_copy(outs[p], out_hbm.at[pl.ds(0, G)], sos[p]).wait()


_sc_pool = pl.kernel(
    _sc_pool_body,
    out_type=jax.ShapeDtypeStruct((CB, EMB), jnp.float32),
    mesh=plsc.VectorSubcoreMesh(core_axis_name="c", subcore_axis_name="s"),
    scratch_types=(
        [pltpu.VMEM((GPW, G * S), jnp.int32),
         pltpu.VMEM((NBUF * G * S, EMB), jnp.float32),
         pltpu.VMEM((NBUF * G, EMB), jnp.float32)]
        + [pltpu.SemaphoreType.DMA for _ in range(2 * NBUF)]
    ),
)


def _mlp_body(x_ref, w1_ref, b1_ref, w2_ref, b2_ref, o_ref):
    x = x_ref[...]
    h = jnp.dot(x, w1_ref[...], preferred_element_type=jnp.float32)
    h = jnp.maximum(h + b1_ref[...], 0.0)
    o = jnp.dot(h, w2_ref[...], preferred_element_type=jnp.float32)
    o_ref[...] = jax.nn.sigmoid(o + b2_ref[...])[:, :1]


def _mlp(x, w1, b1, w2, b2):
    BM = 2048
    grid = (CB // BM,)
    return pl.pallas_call(
        _mlp_body,
        out_shape=jax.ShapeDtypeStruct((CB, 1), jnp.float32),
        grid=grid,
        in_specs=[
            pl.BlockSpec((BM, EMB), lambda i: (i, 0)),
            pl.BlockSpec((EMB, HID), lambda i: (0, 0)),
            pl.BlockSpec((1, HID), lambda i: (0, 0)),
            pl.BlockSpec((HID, 128), lambda i: (0, 0)),
            pl.BlockSpec((1, 128), lambda i: (0, 0)),
        ],
        out_specs=pl.BlockSpec((BM, 1), lambda i: (i, 0)),
    )(x, w1, b1, w2, b2)


def kernel(inputs, table, W1, b1, W2, b2):
    idx = inputs.astype(jnp.int32).reshape(B // G, G * S)
    w2p = jnp.pad(W2, ((0, 0), (0, 128 - W2.shape[1])))
    b2p = jnp.pad(b2, (0, 128 - b2.shape[0])).reshape(1, 128)
    b1r = b1.reshape(1, HID)
    outs = []
    for c in range(NCHUNK):
        pooled = _sc_pool(idx[c * CB // G:(c + 1) * CB // G], table)
        outs.append(_mlp(pooled, W1, b1r, w2p, b2p))
    return jnp.concatenate(outs, axis=0)


# speedup vs baseline: 1.0310x; 1.0004x over previous
import functools

import jax
import jax.numpy as jnp
from jax import lax
from jax.experimental import pallas as pl
from jax.experimental.pallas import tpu as pltpu
from jax.experimental.pallas import tpu_sc as plsc

B = 16384
S = 50
EMB = 128
HID = 256

NC, NS = 2, 16
NW = NC * NS
NCHUNK = 1
CB = B // NCHUNK
EPW = CB // NW
G = 1
NGROUPS = EPW // G
GPW = NGROUPS
NVR = EMB // 16
RU = 10
NBUF = 8
DEPTH = 7


def _sc_pool_body(idx_hbm, table_hbm, out_hbm, idx_all, rows_all, outs_all,
                  *sems):
    rows = [rows_all.at[pl.ds(p * G * S, G * S)] for p in range(NBUF)]
    outs = [outs_all.at[pl.ds(p * G, G)] for p in range(NBUF)]
    sgs = sems[0:NBUF]
    sos = sems[NBUF:2 * NBUF]
    wid = lax.axis_index("s") * NC + lax.axis_index("c")
    ebase = wid * EPW
    pltpu.sync_copy(idx_hbm.at[pl.ds(wid * GPW, GPW)], idx_all)

    def prefetch(g, p):
        pltpu.async_copy(table_hbm.at[idx_all.at[g]], rows[p], sgs[p])

    def consume(i, g, p):
        pltpu.make_async_copy(
            table_hbm.at[idx_all.at[g]], rows[p], sgs[p]).wait()

        @pl.when(i >= 1)
        def _():
            pltpu.make_async_copy(
                outs[p], out_hbm.at[pl.ds(0, G)], sos[p]).wait()

        for e in range(G):
            def row_body(rr, accs, e=e):
                base = (p * G + e) * S + rr * RU
                for k in range(RU):
                    accs = tuple(
                        accs[v] + rows_all[base + k, pl.ds(16 * v, 16)]
                        for v in range(NVR))
                return accs
            accs = lax.fori_loop(
                0, S // RU, row_body,
                tuple(jnp.zeros((16,), jnp.float32) for _ in range(NVR)))
            for v in range(NVR):
                outs_all[p * G + e, pl.ds(16 * v, 16)] = accs[v] * (1.0 / S)
        pltpu.async_copy(outs[p], out_hbm.at[pl.ds(ebase + g * G, G)], sos[p])

    for d in range(DEPTH):
        prefetch(d, d)

    def block(i, carry):
        for p in range(NBUF):
            g = NBUF * i + p

            @pl.when(g + DEPTH < NGROUPS)
            def _(p=p, g=g):
                prefetch(g + DEPTH, (p + DEPTH) % NBUF)

            consume(i, g, p)
        return carry

    lax.fori_loop(0, NGROUPS // NBUF, block, 0)
    for p in range(NBUF):
        pltpu.make_async_copy(outs[p], out_hbm.at[pl.ds(0, G)], sos[p]).wait()


_sc_pool = pl.kernel(
    _sc_pool_body,
    out_type=jax.ShapeDtypeStruct((CB, EMB), jnp.float32),
    mesh=plsc.VectorSubcoreMesh(core_axis_name="c", subcore_axis_name="s"),
    scratch_types=(
        [pltpu.VMEM((GPW, G * S), jnp.int32),
         pltpu.VMEM((NBUF * G * S, EMB), jnp.float32),
         pltpu.VMEM((NBUF * G, EMB), jnp.float32)]
        + [pltpu.SemaphoreType.DMA for _ in range(2 * NBUF)]
    ),
)


def _mlp_body(x_ref, w1_ref, b1_ref, w2_ref, b2_ref, o_ref):
    x = x_ref[...]
    h = jnp.dot(x, w1_ref[...], preferred_element_type=jnp.float32)
    h = jnp.maximum(h + b1_ref[...], 0.0)
    o = jnp.dot(h, w2_ref[...], preferred_element_type=jnp.float32)
    o_ref[...] = jax.nn.sigmoid(o + b2_ref[...])[:, :1]


def _mlp(x, w1, b1, w2, b2):
    BM = 2048
    grid = (CB // BM,)
    return pl.pallas_call(
        _mlp_body,
        out_shape=jax.ShapeDtypeStruct((CB, 1), jnp.float32),
        grid=grid,
        in_specs=[
            pl.BlockSpec((BM, EMB), lambda i: (i, 0)),
            pl.BlockSpec((EMB, HID), lambda i: (0, 0)),
            pl.BlockSpec((1, HID), lambda i: (0, 0)),
            pl.BlockSpec((HID, 128), lambda i: (0, 0)),
            pl.BlockSpec((1, 128), lambda i: (0, 0)),
        ],
        out_specs=pl.BlockSpec((BM, 1), lambda i: (i, 0)),
    )(x, w1, b1, w2, b2)


def kernel(inputs, table, W1, b1, W2, b2):
    idx = inputs.astype(jnp.int32).reshape(B // G, G * S)
    w2p = jnp.pad(W2, ((0, 0), (0, 128 - W2.shape[1])))
    b2p = jnp.pad(b2, (0, 128 - b2.shape[0])).reshape(1, 128)
    b1r = b1.reshape(1, HID)
    outs = []
    for c in range(NCHUNK):
        pooled = _sc_pool(idx[c * CB // G:(c + 1) * CB // G], table)
        outs.append(_mlp(pooled, W1, b1r, w2p, b2p))
    return outs[0] if NCHUNK == 1 else jnp.concatenate(outs, axis=0)
